# baseline (speedup 1.0000x reference)
import functools

import jax
import jax.numpy as jnp
from jax import lax
from jax.experimental import pallas as pl
from jax.experimental.pallas import tpu as pltpu
from jax.experimental.pallas import tpu_sc as plsc

NUM_SC_CORES = 2
NUM_SUBCORES = 16
NW = NUM_SC_CORES * NUM_SUBCORES
BATCH = 16384
EMBED = 64
B_PER_W = BATCH // NW
CHUNK = 128
N_CHUNKS = B_PER_W // CHUNK


def _gather_body(ut_hbm, it_hbm, uidx_hbm, iidx_hbm, ue_hbm, ie_hbm,
                 uidx_v, iidx_v, urows_v, irows_v, sem):
    wid = lax.axis_index("s") * NUM_SC_CORES + lax.axis_index("c")
    base = wid * B_PER_W
    pltpu.sync_copy(uidx_hbm.at[wid], uidx_v)
    pltpu.sync_copy(iidx_hbm.at[wid], iidx_v)
    copies = []
    for j in range(N_CHUNKS):
        copies.append(pltpu.async_copy(
            ut_hbm.at[uidx_v.at[j]], urows_v.at[pl.ds(j * CHUNK, CHUNK)], sem))
        copies.append(pltpu.async_copy(
            it_hbm.at[iidx_v.at[j]], irows_v.at[pl.ds(j * CHUNK, CHUNK)], sem))
    for c in copies:
        c.wait()
    pltpu.sync_copy(urows_v, ue_hbm.at[pl.ds(base, B_PER_W)])
    pltpu.sync_copy(irows_v, ie_hbm.at[pl.ds(base, B_PER_W)])


def _sc_gather(user_table, item_table, uidx, iidx):
    mesh = plsc.VectorSubcoreMesh(core_axis_name="c", subcore_axis_name="s")
    k = pl.kernel(
        _gather_body,
        out_type=(
            jax.ShapeDtypeStruct((BATCH, EMBED), jnp.float32),
            jax.ShapeDtypeStruct((BATCH, EMBED), jnp.float32),
        ),
        mesh=mesh,
        scratch_types=[
            pltpu.VMEM((N_CHUNKS, CHUNK), jnp.int32),
            pltpu.VMEM((N_CHUNKS, CHUNK), jnp.int32),
            pltpu.VMEM((B_PER_W, EMBED), jnp.float32),
            pltpu.VMEM((B_PER_W, EMBED), jnp.float32),
            pltpu.SemaphoreType.DMA,
        ],
        compiler_params=pltpu.CompilerParams(use_tc_tiling_on_sc=False),
    )
    return k(user_table, item_table, uidx, iidx)


BLK = 2048


def _mlp_body(ue, ie, w1u, w1i, b1, w2, b2, w3, b3, w4, b4, out):
    h = jnp.dot(ue[...], w1u[...], preferred_element_type=jnp.float32)
    h = h + jnp.dot(ie[...], w1i[...], preferred_element_type=jnp.float32)
    h = jnp.maximum(h + b1[...], 0.0)
    h = jnp.maximum(
        jnp.dot(h, w2[...], preferred_element_type=jnp.float32) + b2[...], 0.0)
    h = jnp.maximum(
        jnp.dot(h, w3[...], preferred_element_type=jnp.float32) + b3[...], 0.0)
    out[...] = jnp.sum(h * w4[...], axis=1) + b4[0, 0]


def _tc_mlp(ue, ie, w1u, w1i, b1, w2, b2, w3, b3, w4, b4):
    grid = (BATCH // BLK,)
    full = lambda i: (0, 0)
    return pl.pallas_call(
        _mlp_body,
        grid=grid,
        in_specs=[
            pl.BlockSpec((BLK, EMBED), lambda i: (i, 0)),
            pl.BlockSpec((BLK, EMBED), lambda i: (i, 0)),
            pl.BlockSpec((EMBED, 128), full),
            pl.BlockSpec((EMBED, 128), full),
            pl.BlockSpec((1, 128), full),
            pl.BlockSpec((128, 64), full),
            pl.BlockSpec((1, 64), full),
            pl.BlockSpec((64, 32), full),
            pl.BlockSpec((1, 32), full),
            pl.BlockSpec((1, 32), full),
            pl.BlockSpec((1, 1), full),
        ],
        out_specs=pl.BlockSpec((BLK,), lambda i: (i,)),
        out_shape=jax.ShapeDtypeStruct((BATCH,), jnp.float32),
    )(ue, ie, w1u, w1i, b1, w2, b2, w3, b3, w4, b4)


def kernel(user_indices, item_indices, user_table, item_table,
           W1, b1, W2, b2, W3, b3, W4, b4):
    uidx = user_indices.astype(jnp.int32).reshape(NW, N_CHUNKS, CHUNK)
    iidx = item_indices.astype(jnp.int32).reshape(NW, N_CHUNKS, CHUNK)
    ue, ie = _sc_gather(user_table, item_table, uidx, iidx)
    w1t = W1.T
    w1u = w1t[:EMBED]
    w1i = w1t[EMBED:]
    return _tc_mlp(
        ue, ie, w1u, w1i, b1.reshape(1, 128),
        W2.T, b2.reshape(1, 64),
        W3.T, b3.reshape(1, 32),
        W4, b4.reshape(1, 1))


# speedup vs baseline: 2.0074x; 2.0074x over previous
import functools

import jax
import jax.numpy as jnp
from jax import lax
from jax.experimental import pallas as pl
from jax.experimental.pallas import tpu as pltpu
from jax.experimental.pallas import tpu_sc as plsc

NUM_SC_CORES = 2
NUM_SUBCORES = 16
NW = NUM_SC_CORES * NUM_SUBCORES
BATCH = 16384
EMBED = 64
NROWS = 1000000
SPLIT = 503808
NBLK_HALF = 123
B_PER_W = BATCH // NW
CHUNK = 128
N_CHUNKS = B_PER_W // CHUNK

RW = 4096


def _repack_body(tl, tr, eye, out):
    left = lax.dot_general(tl[...], eye[...], (((0,), (0,)), ((), ())),
                           preferred_element_type=jnp.float32)
    right = lax.dot_general(tr[...], eye[...], (((0,), (0,)), ((), ())),
                            preferred_element_type=jnp.float32)
    out[...] = jnp.concatenate([left, right], axis=1)


def _tc_repack(tableT, eye):
    grid = (NBLK_HALF,)
    return pl.pallas_call(
        _repack_body,
        grid=grid,
        in_specs=[
            pl.BlockSpec((EMBED, RW), lambda i: (0, i)),
            pl.BlockSpec((EMBED, RW),
                         lambda i: (0, jnp.minimum(i + NBLK_HALF,
                                                   (NROWS - 1) // RW))),
            pl.BlockSpec((EMBED, EMBED), lambda i: (0, 0)),
        ],
        out_specs=pl.BlockSpec((RW, 2 * EMBED), lambda i: (i, 0)),
        out_shape=jax.ShapeDtypeStruct((SPLIT, 2 * EMBED), jnp.float32),
    )(tableT, tableT, eye)


def _gather_body(ut_hbm, it_hbm, uidx_hbm, iidx_hbm, ue_hbm, ie_hbm,
                 idx_v, rows_v, sem):
    wid = lax.axis_index("s") * NUM_SC_CORES + lax.axis_index("c")
    base = wid * B_PER_W
    pltpu.sync_copy(uidx_hbm.at[wid], idx_v)
    copies = [
        pltpu.async_copy(ut_hbm.at[idx_v.at[j]],
                         rows_v.at[pl.ds(j * CHUNK, CHUNK)], sem)
        for j in range(N_CHUNKS)
    ]
    for c in copies:
        c.wait()
    pltpu.sync_copy(rows_v, ue_hbm.at[pl.ds(base, B_PER_W)])

    pltpu.sync_copy(iidx_hbm.at[wid], idx_v)
    copies = [
        pltpu.async_copy(it_hbm.at[idx_v.at[j]],
                         rows_v.at[pl.ds(j * CHUNK, CHUNK)], sem)
        for j in range(N_CHUNKS)
    ]
    for c in copies:
        c.wait()
    pltpu.sync_copy(rows_v, ie_hbm.at[pl.ds(base, B_PER_W)])


def _sc_gather(user_packed, item_packed, uidx, iidx):
    mesh = plsc.VectorSubcoreMesh(core_axis_name="c", subcore_axis_name="s")
    k = pl.kernel(
        _gather_body,
        out_type=(
            jax.ShapeDtypeStruct((BATCH, 2 * EMBED), jnp.float32),
            jax.ShapeDtypeStruct((BATCH, 2 * EMBED), jnp.float32),
        ),
        mesh=mesh,
        scratch_types=[
            pltpu.VMEM((N_CHUNKS, CHUNK), jnp.int32),
            pltpu.VMEM((B_PER_W, 2 * EMBED), jnp.float32),
            pltpu.SemaphoreType.DMA,
        ],
    )
    return k(user_packed, item_packed, uidx, iidx)


BLK = 2048


def _mlp_body(gu, gi, pu, pi, w1u, w1i, b1, w2, b2, w3, b3, w4, b4, out):
    ua = jnp.dot(gu[:, :EMBED], w1u[...], preferred_element_type=jnp.float32)
    ub = jnp.dot(gu[:, EMBED:], w1u[...], preferred_element_type=jnp.float32)
    ia = jnp.dot(gi[:, :EMBED], w1i[...], preferred_element_type=jnp.float32)
    ib = jnp.dot(gi[:, EMBED:], w1i[...], preferred_element_type=jnp.float32)
    h = jnp.where(pu[...] > 0.5, ub, ua) + jnp.where(pi[...] > 0.5, ib, ia)
    h = jnp.maximum(h + b1[...], 0.0)
    h = jnp.maximum(
        jnp.dot(h, w2[...], preferred_element_type=jnp.float32) + b2[...], 0.0)
    h = jnp.maximum(
        jnp.dot(h, w3[...], preferred_element_type=jnp.float32) + b3[...], 0.0)
    out[...] = jnp.sum(h * w4[...], axis=1) + b4[0, 0]


def _tc_mlp(gu, gi, pu, pi, w1u, w1i, b1, w2, b2, w3, b3, w4, b4):
    grid = (BATCH // BLK,)
    full = lambda i: (0, 0)
    return pl.pallas_call(
        _mlp_body,
        grid=grid,
        in_specs=[
            pl.BlockSpec((BLK, 2 * EMBED), lambda i: (i, 0)),
            pl.BlockSpec((BLK, 2 * EMBED), lambda i: (i, 0)),
            pl.BlockSpec((BLK, 1), lambda i: (i, 0)),
            pl.BlockSpec((BLK, 1), lambda i: (i, 0)),
            pl.BlockSpec((EMBED, 128), full),
            pl.BlockSpec((EMBED, 128), full),
            pl.BlockSpec((1, 128), full),
            pl.BlockSpec((128, 64), full),
            pl.BlockSpec((1, 64), full),
            pl.BlockSpec((64, 32), full),
            pl.BlockSpec((1, 32), full),
            pl.BlockSpec((1, 32), full),
            pl.BlockSpec((1, 1), full),
        ],
        out_specs=pl.BlockSpec((BLK,), lambda i: (i,)),
        out_shape=jax.ShapeDtypeStruct((BATCH,), jnp.float32),
    )(gu, gi, pu, pi, w1u, w1i, b1, w2, b2, w3, b3, w4, b4)


def kernel(user_indices, item_indices, user_table, item_table,
           W1, b1, W2, b2, W3, b3, W4, b4):
    ui = user_indices.astype(jnp.int32)
    ii = item_indices.astype(jnp.int32)
    urow = jnp.where(ui >= SPLIT, ui - SPLIT, ui).reshape(NW, N_CHUNKS, CHUNK)
    irow = jnp.where(ii >= SPLIT, ii - SPLIT, ii).reshape(NW, N_CHUNKS, CHUNK)
    pu = (ui >= SPLIT).astype(jnp.float32).reshape(BATCH, 1)
    pi = (ii >= SPLIT).astype(jnp.float32).reshape(BATCH, 1)

    eye = jnp.eye(EMBED, dtype=jnp.float32)
    user_packed = _tc_repack(user_table.T, eye)
    item_packed = _tc_repack(item_table.T, eye)
    gu, gi = _sc_gather(user_packed, item_packed, urow, irow)

    w1t = W1.T
    return _tc_mlp(
        gu, gi, pu, pi, w1t[:EMBED], w1t[EMBED:], b1.reshape(1, 128),
        W2.T, b2.reshape(1, 64),
        W3.T, b3.reshape(1, 32),
        W4, b4.reshape(1, 1))


# speedup vs baseline: 2.9899x; 1.4894x over previous
import functools

import jax
import jax.numpy as jnp
from jax import lax
from jax.experimental import pallas as pl
from jax.experimental.pallas import tpu as pltpu
from jax.experimental.pallas import tpu_sc as plsc

NUM_SC_CORES = 2
NUM_SUBCORES = 16
NW = NUM_SC_CORES * NUM_SUBCORES
BATCH = 16384
EMBED = 64
NROWS = 1000000
RW = 4096
NBLK4 = 62
SPLIT4 = NBLK4 * RW
LASTBLK = (NROWS - 1) // RW
B_PER_W = BATCH // NW
CHUNK = 128
N_CHUNKS = B_PER_W // CHUNK


def _rne16(t):
    u = lax.bitcast_convert_type(t, jnp.uint32)
    return (u + jnp.uint32(0x7FFF) + ((u >> 16) & jnp.uint32(1))) >> 16


def _repack_body(t0, t1, t2, t3, out):
    z01 = (_rne16(t0[...]) | (_rne16(t1[...]) << 16)).T
    z23 = (_rne16(t2[...]) | (_rne16(t3[...]) << 16)).T
    out[...] = jnp.concatenate([z01, z23], axis=1)


def _tc_repack(tableT):
    def mk(e):
        return pl.BlockSpec(
            (EMBED, RW),
            lambda i, e=e: (0, jnp.minimum(i + e * NBLK4, LASTBLK)))
    return pl.pallas_call(
        _repack_body,
        grid=(NBLK4,),
        in_specs=[mk(0), mk(1), mk(2), mk(3)],
        out_specs=pl.BlockSpec((RW, 128), lambda i: (i, 0)),
        out_shape=jax.ShapeDtypeStruct((SPLIT4, 128), jnp.uint32),
    )(tableT, tableT, tableT, tableT)


def _gather_body(ut_hbm, it_hbm, uidx_hbm, iidx_hbm, ue_hbm, ie_hbm,
                 idx_v, rows_v, sem):
    wid = lax.axis_index("s") * NUM_SC_CORES + lax.axis_index("c")
    base = wid * B_PER_W
    pltpu.sync_copy(uidx_hbm.at[wid], idx_v)
    copies = [
        pltpu.async_copy(ut_hbm.at[idx_v.at[j]],
                         rows_v.at[pl.ds(j * CHUNK, CHUNK)], sem)
        for j in range(N_CHUNKS)
    ]
    for c in copies:
        c.wait()
    pltpu.sync_copy(rows_v, ue_hbm.at[pl.ds(base, B_PER_W)])

    pltpu.sync_copy(iidx_hbm.at[wid], idx_v)
    copies = [
        pltpu.async_copy(it_hbm.at[idx_v.at[j]],
                         rows_v.at[pl.ds(j * CHUNK, CHUNK)], sem)
        for j in range(N_CHUNKS)
    ]
    for c in copies:
        c.wait()
    pltpu.sync_copy(rows_v, ie_hbm.at[pl.ds(base, B_PER_W)])


def _sc_gather(user_packed, item_packed, uidx, iidx):
    mesh = plsc.VectorSubcoreMesh(core_axis_name="c", subcore_axis_name="s")
    k = pl.kernel(
        _gather_body,
        out_type=(
            jax.ShapeDtypeStruct((BATCH, 128), jnp.uint32),
            jax.ShapeDtypeStruct((BATCH, 128), jnp.uint32),
        ),
        mesh=mesh,
        scratch_types=[
            pltpu.VMEM((N_CHUNKS, CHUNK), jnp.int32),
            pltpu.VMEM((B_PER_W, 128), jnp.uint32),
            pltpu.SemaphoreType.DMA,
        ],
    )
    return k(user_packed, item_packed, uidx, iidx)


BLK = 2048


def _select4(g_u32, e0, e1):
    gh = jnp.where(e1 > 0.5, g_u32[:, EMBED:], g_u32[:, :EMBED])
    bits = jnp.where(e0 > 0.5, gh & jnp.uint32(0xFFFF0000), gh << 16)
    return lax.bitcast_convert_type(bits, jnp.float32)


def _mlp_body(gu, gi, eu0, eu1, ei0, ei1,
              w1u, w1i, b1, w2, b2, w3, b3, w4, b4, out):
    xu = _select4(gu[...], eu0[...], eu1[...])
    xi = _select4(gi[...], ei0[...], ei1[...])
    h = (jnp.dot(xu, w1u[...], preferred_element_type=jnp.float32)
         + jnp.dot(xi, w1i[...], preferred_element_type=jnp.float32))
    h = jnp.maximum(h + b1[...], 0.0)
    h = jnp.maximum(
        jnp.dot(h, w2[...], preferred_element_type=jnp.float32) + b2[...], 0.0)
    h = jnp.maximum(
        jnp.dot(h, w3[...], preferred_element_type=jnp.float32) + b3[...], 0.0)
    out[...] = jnp.sum(h * w4[...], axis=1) + b4[0, 0]


def _tc_mlp(gu, gi, eu0, eu1, ei0, ei1, w1u, w1i, b1, w2, b2, w3, b3, w4, b4):
    grid = (BATCH // BLK,)
    full = lambda i: (0, 0)
    col = lambda i: (i, 0)
    return pl.pallas_call(
        _mlp_body,
        grid=grid,
        in_specs=[
            pl.BlockSpec((BLK, 128), col),
            pl.BlockSpec((BLK, 128), col),
            pl.BlockSpec((BLK, 1), col),
            pl.BlockSpec((BLK, 1), col),
            pl.BlockSpec((BLK, 1), col),
            pl.BlockSpec((BLK, 1), col),
            pl.BlockSpec((EMBED, 128), full),
            pl.BlockSpec((EMBED, 128), full),
            pl.BlockSpec((1, 128), full),
            pl.BlockSpec((128, 64), full),
            pl.BlockSpec((1, 64), full),
            pl.BlockSpec((64, 32), full),
            pl.BlockSpec((1, 32), full),
            pl.BlockSpec((1, 32), full),
            pl.BlockSpec((1, 1), full),
        ],
        out_specs=pl.BlockSpec((BLK,), lambda i: (i,)),
        out_shape=jax.ShapeDtypeStruct((BATCH,), jnp.float32),
    )(gu, gi, eu0, eu1, ei0, ei1, w1u, w1i, b1, w2, b2, w3, b3, w4, b4)


def kernel(user_indices, item_indices, user_table, item_table,
           W1, b1, W2, b2, W3, b3, W4, b4):
    ui = user_indices.astype(jnp.int32)
    ii = item_indices.astype(jnp.int32)
    eu = ui // SPLIT4
    ei = ii // SPLIT4
    urow = (ui - eu * SPLIT4).reshape(NW, N_CHUNKS, CHUNK)
    irow = (ii - ei * SPLIT4).reshape(NW, N_CHUNKS, CHUNK)
    eu0 = (eu & 1).astype(jnp.float32).reshape(BATCH, 1)
    eu1 = (eu >= 2).astype(jnp.float32).reshape(BATCH, 1)
    ei0 = (ei & 1).astype(jnp.float32).reshape(BATCH, 1)
    ei1 = (ei >= 2).astype(jnp.float32).reshape(BATCH, 1)

    user_packed = _tc_repack(user_table.T)
    item_packed = _tc_repack(item_table.T)
    gu, gi = _sc_gather(user_packed, item_packed, urow, irow)

    w1t = W1.T
    return _tc_mlp(
        gu, gi, eu0, eu1, ei0, ei1,
        w1t[:EMBED], w1t[EMBED:], b1.reshape(1, 128),
        W2.T, b2.reshape(1, 64),
        W3.T, b3.reshape(1, 32),
        W4, b4.reshape(1, 1))
